# Initial kernel scaffold; baseline (speedup 1.0000x reference)
import jax
import jax.numpy as jnp
from jax.experimental import pallas as pl


def kernel(boxes, scores):
    raise NotImplementedError("write your pallas kernel here")


# speedup vs baseline: 54.5687x; 54.5687x over previous
import functools

import jax
import jax.numpy as jnp
from jax import lax
from jax.experimental import pallas as pl
from jax.experimental.pallas import tpu as pltpu
from jax.experimental.pallas import tpu_sc as plsc

N_BOX = 5000
N_CLS = 80
N_FLAT = N_BOX * N_CLS
N_PAD = 401408
CHUNK = N_PAD // 32
CROWS = CHUNK // 128
N_CAND = 1000
CAP = 1024
ROWW = 16
I32_MIN = -2147483648
NEG_INF_BITS = -8388608
SCORE_THR = 0.001
IOU_THR = 0.7
OFFS = 4096.0


def _key_from_scores(s):
    sp = jnp.where(s > SCORE_THR, s, jnp.float32(-1.0))
    b = lax.bitcast_convert_type(sp, jnp.int32)
    k = jnp.where(b >= 0, b, I32_MIN - b)
    raw = lax.bitcast_convert_type(s, jnp.int32)
    return jnp.where(raw == NEG_INF_BITS, I32_MIN, k)


def _s1_body(s_ref, o_ref, k_ref):
    k_ref[...] = _key_from_scores(s_ref[...])

    def bit(t, u):
        candu = u | lax.shift_left(jnp.int32(1), jnp.int32(31) - t)
        cand_signed = candu ^ I32_MIN
        cnt = jnp.sum((k_ref[...] >= cand_signed).astype(jnp.int32))
        return jnp.where(cnt >= N_CAND, candu, u)

    u = lax.fori_loop(0, 32, bit, jnp.int32(0))
    kstar = u ^ I32_MIN
    cnt_gt = jnp.sum((k_ref[...] > kstar).astype(jnp.int32))
    need_eq = N_CAND - cnt_gt

    one = jnp.float32(1.0)
    zero = jnp.float32(0.0)
    k = k_ref[...]
    rs_gt = jnp.sum(jnp.where(k > kstar, one, zero), axis=1, keepdims=True)
    rs_eq = jnp.sum(jnp.where(k == kstar, one, zero), axis=1, keepdims=True)
    cc = lax.broadcasted_iota(jnp.int32, (128, N_PAD // 128), 0)
    rr = lax.broadcasted_iota(jnp.int32, (128, N_PAD // 128), 1)
    S = jnp.where((cc < 32) & (rr >= cc * CROWS) & (rr < (cc + 1) * CROWS),
                  one, zero)
    dn = (((0,), (1,)), ((), ()))
    hi = lax.Precision.HIGHEST
    cgt_row = lax.dot_general(rs_gt, S, dn, precision=hi,
                              preferred_element_type=jnp.float32)
    ceq_row = lax.dot_general(rs_eq, S, dn, precision=hi,
                              preferred_element_type=jnp.float32)
    ca = lax.broadcasted_iota(jnp.int32, (128, 128), 0)
    cb = lax.broadcasted_iota(jnp.int32, (128, 128), 1)
    L = jnp.where((ca < cb) & (cb < 32), one, zero)
    dn2 = (((1,), (0,)), ((), ()))
    pgt_row = lax.dot_general(cgt_row, L, dn2, precision=hi,
                              preferred_element_type=jnp.float32)
    peq_row = lax.dot_general(ceq_row, L, dn2, precision=hi,
                              preferred_element_type=jnp.float32)

    r_io = lax.broadcasted_iota(jnp.int32, (8, 128), 0)
    lane = lax.broadcasted_iota(jnp.int32, (8, 128), 1)
    out = jnp.where(lane == 0, kstar,
                    jnp.where(lane == 1, cnt_gt,
                              jnp.where(lane == 2, need_eq, 0)))
    out = jnp.where(r_io == 1, cgt_row.astype(jnp.int32), out)
    out = jnp.where(r_io == 2, pgt_row.astype(jnp.int32), out)
    out = jnp.where(r_io == 3, ceq_row.astype(jnp.int32), out)
    out = jnp.where(r_io == 4, peq_row.astype(jnp.int32), out)
    o_ref[...] = out


def _stage1(spad2d):
    return pl.pallas_call(
        _s1_body,
        out_shape=jax.ShapeDtypeStruct((8, 128), jnp.int32),
        scratch_shapes=[pltpu.VMEM((N_PAD // 128, 128), jnp.int32)],
    )(spad2d)


def _iota16():
    return lax.broadcasted_iota(jnp.int32, (16,), 0)


def _s2_body(scores_hbm, boxes_hbm, scal_hbm,
             out_gt, out_eq,
             chunk_v, boxes_v, scal_v,
             bgt_i, bgt_s, beq_i, beq_s,
             row_gt, row_eq):
    wid = lax.axis_index("s") * 2 + lax.axis_index("c")
    base = wid * CHUNK
    pltpu.sync_copy(scores_hbm.at[pl.ds(base, CHUNK)], chunk_v)
    pltpu.sync_copy(boxes_hbm, boxes_v)
    pltpu.sync_copy(scal_hbm, scal_v)

    lanes = _iota16()

    def lane_of(vec, j):
        return jnp.sum(jnp.where(lanes == j, vec, 0))

    head = scal_v[pl.ds(0, 16)]
    kstar = lane_of(head, 0)
    need_eq = lane_of(head, 2)
    my_cgt = lane_of(
        plsc.load_gather(scal_v, [jnp.full((16,), 128, jnp.int32) + wid]), 0)
    gt_pre = lane_of(
        plsc.load_gather(scal_v, [jnp.full((16,), 256, jnp.int32) + wid]), 0)
    my_ceq = lane_of(
        plsc.load_gather(scal_v, [jnp.full((16,), 384, jnp.int32) + wid]), 0)
    eq_pre = lane_of(
        plsc.load_gather(scal_v, [jnp.full((16,), 512, jnp.int32) + wid]), 0)
    taken_eq = jnp.clip(need_eq - eq_pre, 0, my_ceq)
    eq_off = jnp.minimum(eq_pre, need_eq)

    z16 = jnp.zeros((16,), jnp.int32)

    def comp_body(j, carry):
        ngt, neq = carry
        v = chunk_v[pl.ds(j * 16, 16)]
        sp = jnp.where(v > SCORE_THR, v, jnp.float32(-1.0))
        k = _key_from_scores(v)
        fidx = base + j * 16 + lanes
        mgt = k > kstar
        meq = k == kstar
        igt = jnp.where(mgt, 1, 0).astype(jnp.int32)
        ieq = jnp.where(meq, 1, 0).astype(jnp.int32)
        pos_gt = ngt + plsc.cumsum(igt) - igt
        plsc.store_scatter(bgt_i, [pos_gt], fidx, mask=mgt)
        plsc.store_scatter(bgt_s, [pos_gt], sp, mask=mgt)
        rel = neq + plsc.cumsum(ieq) - ieq
        meq_st = meq & (rel < taken_eq)
        plsc.store_scatter(beq_i, [rel], fidx, mask=meq_st)
        plsc.store_scatter(beq_s, [rel], sp, mask=meq_st)
        ngt = ngt + plsc.all_reduce_population_count(mgt)
        neq = neq + plsc.all_reduce_population_count(meq)
        return ngt, neq

    lax.fori_loop(0, CHUNK // 16, comp_body, (z16, z16))

    def emit(cnt_s, off_s, idx_buf, sc_buf, row_buf, out_hbm):
        ngrp = (cnt_s + 15) // 16

        def g_body(g, _):
            fidx = idx_buf[pl.ds(g * 16, 16)]
            sc = sc_buf[pl.ds(g * 16, 16)]
            bi = fidx // N_CLS
            cls = fidx - bi * N_CLS
            bic = jnp.clip(bi, 0, N_BOX - 1)
            x1 = plsc.load_gather(boxes_v, [bic * 4])
            y1 = plsc.load_gather(boxes_v, [bic * 4 + 1])
            x2 = plsc.load_gather(boxes_v, [bic * 4 + 2])
            y2 = plsc.load_gather(boxes_v, [bic * 4 + 3])
            rb = (g * 16 + lanes) * ROWW
            plsc.store_scatter(row_buf, [rb + 0], sc)
            plsc.store_scatter(row_buf, [rb + 1], x1)
            plsc.store_scatter(row_buf, [rb + 2], y1)
            plsc.store_scatter(row_buf, [rb + 3], x2)
            plsc.store_scatter(row_buf, [rb + 4], y2)
            plsc.store_scatter(row_buf, [rb + 5], cls.astype(jnp.float32))
            plsc.store_scatter(row_buf, [rb + 6], fidx.astype(jnp.float32))
            return 0

        lax.fori_loop(0, ngrp, g_body, 0)

        full = cnt_s // 16
        rem = cnt_s - full * 16

        def w_body(g, _):
            src = row_buf.at[pl.ds(g * (16 * ROWW), 16 * ROWW)]
            dst = out_hbm.at[pl.ds(
                pl.multiple_of((off_s + g * 16) * ROWW, 8), 16 * ROWW)]
            pltpu.sync_copy(src, dst)
            return 0

        lax.fori_loop(0, full, w_body, 0)

        def r_body(r, _):
            src = row_buf.at[pl.ds((full * 16 + r) * ROWW, ROWW)]
            dst = out_hbm.at[pl.ds(
                pl.multiple_of((off_s + full * 16 + r) * ROWW, 8), ROWW)]
            pltpu.sync_copy(src, dst)
            return 0

        lax.fori_loop(0, rem, r_body, 0)

    emit(my_cgt, gt_pre, bgt_i, bgt_s, row_gt, out_gt)
    emit(taken_eq, eq_off, beq_i, beq_s, row_eq, out_eq)


def _stage2(spad, boxes_flat, scal_flat):
    mesh = plsc.VectorSubcoreMesh(core_axis_name="c", subcore_axis_name="s")
    f = pl.kernel(
        _s2_body,
        mesh=mesh,
        compiler_params=pltpu.CompilerParams(needs_layout_passes=False),
        out_type=[jax.ShapeDtypeStruct((2048 * ROWW,), jnp.float32),
                  jax.ShapeDtypeStruct((2048 * ROWW,), jnp.float32)],
        scratch_types=[
            pltpu.VMEM((CHUNK,), jnp.float32),
            pltpu.VMEM((N_BOX * 4,), jnp.float32),
            pltpu.VMEM((1024,), jnp.int32),
            pltpu.VMEM((CAP,), jnp.int32),
            pltpu.VMEM((CAP,), jnp.float32),
            pltpu.VMEM((CAP,), jnp.int32),
            pltpu.VMEM((CAP,), jnp.float32),
            pltpu.VMEM((CAP * ROWW,), jnp.float32),
            pltpu.VMEM((CAP * ROWW,), jnp.float32),
        ],
    )
    return f(spad, boxes_flat, scal_flat)


def _outrank(k_me, f_me, i_me, k_ot, f_ot, i_ot):
    return (k_ot > k_me) | ((k_ot == k_me) & ((f_ot < f_me) |
            ((f_ot == f_me) & (i_ot < i_me))))


def _s3_body(gt_ref, eq_ref, scal_ref, fin_ref, nv_ref):
    G = scal_ref[0, 1]
    E = scal_ref[0, 2]
    ii = lax.broadcasted_iota(jnp.int32, (CAP, CAP), 0)
    jj = lax.broadcasted_iota(jnp.int32, (CAP, CAP), 1)
    one = jnp.float32(1.0)
    zero = jnp.float32(0.0)
    rio16 = lax.broadcasted_iota(jnp.int32, (CAP, 16), 0)
    lio16 = lax.broadcasted_iota(jnp.int32, (CAP, 16), 1)
    gt_rows = jnp.where((rio16 < G) & (lio16 < 7), gt_ref[...], zero)
    eq_rows = jnp.where((rio16 < E) & (lio16 < 7), eq_ref[...], zero)
    p_gt = jnp.where((ii == jj) & (ii < G), one, zero)
    p_eq = jnp.where((jj == ii - G) & (ii >= G) & (ii < N_CAND), one, zero)
    hi = lax.Precision.HIGHEST
    C = (jnp.dot(p_gt, gt_rows, precision=hi,
                 preferred_element_type=jnp.float32)
         + jnp.dot(p_eq, eq_rows, precision=hi,
                   preferred_element_type=jnp.float32))

    lane16c = lax.broadcasted_iota(jnp.int32, (16, 1), 0)
    lane16r = lax.broadcasted_iota(jnp.int32, (1, 16), 1)

    def col(f):
        e = jnp.where(lane16c == f, one, zero)
        return jnp.dot(C, e, precision=hi,
                       preferred_element_type=jnp.float32)

    def row(f):
        e = jnp.where(lane16r == f, one, zero)
        return lax.dot_general(e, C, (((1,), (1,)), ((), ())), precision=hi,
                               preferred_element_type=jnp.float32)

    sc_c, sc_r = col(0), row(0)
    x1c, y1c, x2c, y2c = col(1), col(2), col(3), col(4)
    x1r, y1r, x2r, y2r = row(1), row(2), row(3), row(4)
    cls_c, cls_r = col(5), row(5)
    fi_c, fi_r = col(6), row(6)

    offc = cls_c * OFFS
    offr = cls_r * OFFS
    sx1c, sy1c, sx2c, sy2c = x1c + offc, y1c + offc, x2c + offc, y2c + offc
    sx1r, sy1r, sx2r, sy2r = x1r + offr, y1r + offr, x2r + offr, y2r + offr

    def key(s):
        sp = jnp.where(s > SCORE_THR, s, jnp.float32(-1.0))
        b = lax.bitcast_convert_type(sp, jnp.int32)
        return jnp.where(b >= 0, b, I32_MIN - b)

    k_c, k_r = key(sc_c), key(sc_r)
    iic = lax.broadcasted_iota(jnp.int32, (CAP, 1), 0)
    jjr = lax.broadcasted_iota(jnp.int32, (1, CAP), 1)
    cmp1 = _outrank(k_c, fi_c, iic, k_r, fi_r, jjr)
    rank_c = jnp.sum(jnp.where(cmp1, one, zero), axis=1, keepdims=True)
    cmp2 = _outrank(k_r, fi_r, jjr, k_c, fi_c, iic)
    rank_r = jnp.sum(jnp.where(cmp2, one, zero), axis=0, keepdims=True)

    area_c = jnp.maximum(sx2c - sx1c, zero) * jnp.maximum(sy2c - sy1c, zero)
    area_r = jnp.maximum(sx2r - sx1r, zero) * jnp.maximum(sy2r - sy1r, zero)
    w = jnp.maximum(jnp.minimum(sx2c, sx2r) - jnp.maximum(sx1c, sx1r), zero)
    h = jnp.maximum(jnp.minimum(sy2c, sy2r) - jnp.maximum(sy1c, sy1r), zero)
    inter = w * h
    union = area_c + area_r - inter
    iou = inter / jnp.maximum(union, jnp.float32(1e-9))
    msup = jnp.where((iou > IOU_THR) & (rank_c < rank_r), one, zero)

    valid_c = sc_c > zero
    valid_r = sc_r > zero
    k0c = jnp.where(valid_c, one, zero)
    k0r = jnp.where(valid_r, one, zero)

    def cond(carry):
        return carry[2]

    def body(carry):
        kc, kr, _ = carry
        s_r = jnp.dot(kr, msup, precision=hi,
                      preferred_element_type=jnp.float32)
        s_c = lax.dot_general(msup, kc, (((0,), (0,)), ((), ())), precision=hi,
                              preferred_element_type=jnp.float32)
        nkc = jnp.where(valid_c & (s_c < 0.5), one, zero)
        nkr = jnp.where(valid_r & (s_r < 0.5), one, zero)
        changed = jnp.sum(jnp.abs(nkc - kc)) > zero
        return nkc, nkr, changed

    keep_c, keep_r, _ = lax.while_loop(cond, body,
                                       (k0c, k0r, jnp.bool_(True)))

    ks_c = jnp.where(keep_c > 0.5, sc_c, jnp.float32(-1.0))
    ks_r = jnp.where(keep_r > 0.5, sc_r, jnp.float32(-1.0))
    cmp3 = (ks_r > ks_c) | ((ks_r == ks_c) & (rank_r < rank_c))
    r2_c = jnp.sum(jnp.where(cmp3, one, zero), axis=1, keepdims=True)
    cmp4 = (ks_c > ks_r) | ((ks_c == ks_r) & (rank_c < rank_r))
    r2_r = jnp.sum(jnp.where(cmp4, one, zero), axis=0, keepdims=True)

    lane16b = lax.broadcasted_iota(jnp.int32, (CAP, 16), 1)
    combo = jnp.where(lane16b == 7, ks_c, C)
    rio = lax.broadcasted_iota(jnp.int32, (512, CAP), 0).astype(jnp.float32)
    O = jnp.where(rio == r2_r, one, zero)
    R = jnp.dot(O, combo, precision=hi,
                preferred_element_type=jnp.float32)

    vq = R[:, 7:8] > zero
    fin = jnp.where(vq, R, zero)
    lane16o = lax.broadcasted_iota(jnp.int32, (512, 16), 1)
    fin = jnp.where((lane16o == 5) & (~vq), jnp.float32(-1.0), fin)
    fin_ref[...] = fin

    nv = jnp.sum(jnp.where((r2_c < 300.0) & (ks_c > zero), 1, 0)
                 .astype(jnp.int32))
    lane = lax.broadcasted_iota(jnp.int32, (8, 128), 1)
    nv_ref[...] = jnp.where(lane == 0, nv, 0)


def _stage3(gt_rows, eq_rows, scal8):
    return pl.pallas_call(
        _s3_body,
        out_shape=[jax.ShapeDtypeStruct((512, 16), jnp.float32),
                   jax.ShapeDtypeStruct((8, 128), jnp.int32)],
    )(gt_rows, eq_rows, scal8)


def kernel(boxes, scores):
    b0 = boxes[0].reshape(-1)
    flat = scores[0].reshape(-1)
    spad = jnp.concatenate(
        [flat, jnp.full((N_PAD - N_FLAT,), -jnp.inf, jnp.float32)])
    scal8 = _stage1(spad.reshape(N_PAD // 128, 128))
    gt_all, eq_all = _stage2(spad, b0, scal8.reshape(-1))
    gt_rows = gt_all.reshape(2048, ROWW)[:CAP]
    eq_rows = eq_all.reshape(2048, ROWW)[:CAP]
    fin, nv = _stage3(gt_rows, eq_rows, scal8)
    out_boxes = fin[:300, 1:5][None]
    out_scores = fin[:300, 7][None]
    out_labels = fin[:300, 5].astype(jnp.int32)[None]
    n_valid = nv[0, :1]
    return out_boxes, out_scores, out_labels, n_valid
